# baseline (speedup 1.0000x reference)
import functools

import jax
import jax.numpy as jnp
from jax.experimental import pallas as pl
from jax.experimental.pallas import tpu as pltpu

B, C, H, W = 2, 96, 224, 224
F = 96
KH = KW = 3
OH = OW = 111
TH = 32
NT = 4
CHUNK = 576


def _conv_kernel(vals_ref, rows_ref, cols_ref, x1_ref, x2_ref, out_ref, w_ref,
                 sel_ref, *, nchunks):
    b = pl.program_id(0)
    t = pl.program_id(1)

    @pl.when(jnp.logical_and(b == 0, t == 0))
    def _build_weights():
        iota_f = jax.lax.broadcasted_iota(jnp.int32, (F, CHUNK), 0)
        iota_c = jax.lax.broadcasted_iota(jnp.int32, (CHUNK, C), 1)
        for tap in range(KH * KW):
            acc = jnp.zeros((F, C), jnp.float32)
            for ch in range(nchunks):
                rows_c = rows_ref[ch, :][None, :]
                vals_c = vals_ref[ch, :][None, :]
                cols_c = cols_ref[ch, :][:, None]
                sel = jnp.where(rows_c == iota_f, vals_c, 0.0)
                onehot = (cols_c == (iota_c + tap * C)).astype(jnp.float32)
                acc = acc + jnp.dot(sel, onehot,
                                    preferred_element_type=jnp.float32)
            w_ref[tap] = acc
        iw = jax.lax.broadcasted_iota(jnp.int32, (W, W), 0)
        iq = jax.lax.broadcasted_iota(jnp.int32, (W, W), 1)
        sel_ref[...] = jnp.where(
            iq < W // 2, (iw == 2 * iq).astype(jnp.float32),
            (iw == 2 * (iq - W // 2) + 1).astype(jnp.float32))

    def row(l):
        if l < 2 * TH:
            return x1_ref[0, :, l, :]
        return x2_ref[0, :, l - 2 * TH, :]

    sel = sel_ref[...]
    dei = [jnp.dot(row(l), sel, preferred_element_type=jnp.float32)
           for l in range(2 * TH + 1)]

    for r in range(TH):
        acc = jnp.zeros((F, OW), jnp.float32)
        for j in range(KH):
            d = dei[2 * r + j]
            ev = jax.lax.slice(d, (0, 0), (C, W // 2))
            od = jax.lax.slice(d, (0, W // 2), (C, W))
            acc = acc + jnp.dot(w_ref[j], jax.lax.slice(ev, (0, 0), (C, OW)),
                                preferred_element_type=jnp.float32)
            acc = acc + jnp.dot(w_ref[3 + j], jax.lax.slice(od, (0, 0), (C, OW)),
                                preferred_element_type=jnp.float32)
            acc = acc + jnp.dot(w_ref[6 + j], jax.lax.slice(ev, (0, 1), (C, OW + 1)),
                                preferred_element_type=jnp.float32)
        out_ref[0, :, r, :] = acc


def kernel(inputs, values, row_ids, col_idx):
    nnz = values.shape[0]
    nchunks = max(1, -(-nnz // CHUNK))
    pad = nchunks * CHUNK - nnz
    vals2 = jnp.pad(values, (0, pad)).reshape(nchunks, CHUNK)
    rows2 = jnp.pad(row_ids, (0, pad), constant_values=-1).reshape(nchunks, CHUNK)
    cols2 = jnp.pad(col_idx, (0, pad), constant_values=-1).reshape(nchunks, CHUNK)

    grid = (B, NT)
    outT = pl.pallas_call(
        functools.partial(_conv_kernel, nchunks=nchunks),
        grid=grid,
        in_specs=[
            pl.BlockSpec((nchunks, CHUNK), lambda b, t: (0, 0)),
            pl.BlockSpec((nchunks, CHUNK), lambda b, t: (0, 0)),
            pl.BlockSpec((nchunks, CHUNK), lambda b, t: (0, 0)),
            pl.BlockSpec((1, C, 2 * TH, W), lambda b, t: (b, 0, t, 0)),
            pl.BlockSpec((1, C, 8, W),
                         lambda b, t: (b, 0, jnp.minimum(8 * t + 8, 27), 0)),
        ],
        out_specs=pl.BlockSpec((1, F, TH, OW), lambda b, t: (b, 0, t, 0)),
        out_shape=jax.ShapeDtypeStruct((B, F, OH, OW), jnp.float32),
        scratch_shapes=[pltpu.VMEM((KH * KW, F, C), jnp.float32),
                        pltpu.VMEM((W, W), jnp.float32)],
        compiler_params=pltpu.CompilerParams(
            dimension_semantics=("arbitrary", "arbitrary")),
    )(vals2, rows2, cols2, inputs, inputs)
    return jnp.swapaxes(outT, 2, 3)


# speedup vs baseline: 1.0736x; 1.0736x over previous
import functools

import jax
import jax.numpy as jnp
from jax.experimental import pallas as pl
from jax.experimental.pallas import tpu as pltpu

B, C, H, W = 2, 96, 224, 224
F = 96
KH = KW = 3
OH = OW = 111
TH = 16
NT = 7
CHUNK = 576


def _conv_kernel(vals_ref, rows_ref, cols_ref, x1_ref, x2_ref, out_ref, w_ref,
                 sel_ref, *, nchunks):
    b = pl.program_id(0)
    t = pl.program_id(1)

    @pl.when(jnp.logical_and(b == 0, t == 0))
    def _build_weights():
        iota_f = jax.lax.broadcasted_iota(jnp.int32, (F, CHUNK), 0)
        iota_c = jax.lax.broadcasted_iota(jnp.int32, (CHUNK, C), 1)
        accs = [jnp.zeros((F, C), jnp.float32) for _ in range(KH * KW)]
        for ch in range(nchunks):
            rows_c = rows_ref[ch, :][None, :]
            vals_c = vals_ref[ch, :][None, :]
            cols_c = cols_ref[ch]
            sel = jnp.where(rows_c == iota_f, vals_c, 0.0)
            for tap in range(KH * KW):
                onehot = (cols_c == (iota_c + tap * C)).astype(jnp.float32)
                accs[tap] = accs[tap] + jnp.dot(
                    sel, onehot, preferred_element_type=jnp.float32)
        for tap in range(KH * KW):
            w_ref[tap] = accs[tap]
        iw = jax.lax.broadcasted_iota(jnp.int32, (W, W), 0)
        iq = jax.lax.broadcasted_iota(jnp.int32, (W, W), 1)
        sel_ref[...] = jnp.where(
            iq < W // 2, (iw == 2 * iq).astype(jnp.float32),
            (iw == 2 * (iq - W // 2) + 1).astype(jnp.float32))

    def row(l):
        if l < 2 * TH:
            return x1_ref[0, :, l, :]
        return x2_ref[0, :, l - 2 * TH, :]

    sel = sel_ref[...]
    dei = [jnp.dot(row(l), sel, preferred_element_type=jnp.float32)
           for l in range(2 * TH + 1)]

    for r in range(TH):
        acc = jnp.zeros((F, OW), jnp.float32)
        for j in range(KH):
            d = dei[2 * r + j]
            ev = jax.lax.slice(d, (0, 0), (C, W // 2))
            od = jax.lax.slice(d, (0, W // 2), (C, W))
            acc = acc + jnp.dot(w_ref[j], jax.lax.slice(ev, (0, 0), (C, OW)),
                                preferred_element_type=jnp.float32)
            acc = acc + jnp.dot(w_ref[3 + j], jax.lax.slice(od, (0, 0), (C, OW)),
                                preferred_element_type=jnp.float32)
            acc = acc + jnp.dot(w_ref[6 + j], jax.lax.slice(ev, (0, 1), (C, OW + 1)),
                                preferred_element_type=jnp.float32)
        out_ref[0, :, r, :] = acc


def kernel(inputs, values, row_ids, col_idx):
    nnz = values.shape[0]
    nchunks = max(1, -(-nnz // CHUNK))
    pad = nchunks * CHUNK - nnz
    vals2 = jnp.pad(values, (0, pad)).reshape(nchunks, CHUNK)
    rows2 = jnp.pad(row_ids, (0, pad), constant_values=-1).reshape(nchunks, CHUNK)
    cols2 = jnp.pad(col_idx, (0, pad), constant_values=-1).reshape(
        nchunks, CHUNK, 1)

    grid = (B, NT)
    outT = pl.pallas_call(
        functools.partial(_conv_kernel, nchunks=nchunks),
        grid=grid,
        in_specs=[
            pl.BlockSpec((nchunks, CHUNK), lambda b, t: (0, 0)),
            pl.BlockSpec((nchunks, CHUNK), lambda b, t: (0, 0)),
            pl.BlockSpec((nchunks, CHUNK, 1), lambda b, t: (0, 0, 0)),
            pl.BlockSpec((1, C, 2 * TH, W), lambda b, t: (b, 0, t, 0)),
            pl.BlockSpec((1, C, 8, W),
                         lambda b, t: (b, 0, jnp.minimum(4 * t + 4, 27), 0)),
        ],
        out_specs=pl.BlockSpec((1, F, TH, OW), lambda b, t: (b, 0, t, 0)),
        out_shape=jax.ShapeDtypeStruct((B, F, OH, OW), jnp.float32),
        scratch_shapes=[pltpu.VMEM((KH * KW, F, C), jnp.float32),
                        pltpu.VMEM((W, W), jnp.float32)],
        compiler_params=pltpu.CompilerParams(
            dimension_semantics=("arbitrary", "arbitrary")),
    )(vals2, rows2, cols2, inputs, inputs)
    return jnp.swapaxes(outT, 2, 3)


# speedup vs baseline: 1.0803x; 1.0063x over previous
import functools

import jax
import jax.numpy as jnp
from jax import lax
from jax.experimental import pallas as pl
from jax.experimental.pallas import tpu as pltpu
from jax.experimental.pallas import tpu_sc as plsc

B, C, H, W = 2, 96, 224, 224
F = 96
KH = KW = 3
OH = OW = 111
TH = 16
NT = 7
TAPS = KH * KW
WSZ = TAPS * F * C


def _sc_build_weights(nnz_pad):
    info = plsc.get_sparse_core_info()
    ns = info.num_subcores
    stripe = WSZ // ns
    chunk = nnz_pad // ns

    @functools.partial(
        pl.kernel,
        mesh=plsc.VectorSubcoreMesh(core_axis_name="c", subcore_axis_name="s"),
        out_type=jax.ShapeDtypeStruct((WSZ,), jnp.float32),
        scratch_types=[
            pltpu.VMEM((chunk,), jnp.float32),
            pltpu.VMEM((chunk,), jnp.int32),
            pltpu.VMEM((stripe,), jnp.float32),
            pltpu.VMEM_SHARED((WSZ,), jnp.float32),
        ],
    )
    def build(vals_hbm, fidx_hbm, w_hbm, vals_v, fidx_v, zbuf, shared):
        cid = lax.axis_index("c")
        sid = lax.axis_index("s")

        @pl.when(cid == 0)
        def _core0():
            def zero_body(k, carry):
                zbuf[pl.ds(k * 16, 16)] = jnp.zeros((16,), jnp.float32)
                return carry
            lax.fori_loop(0, stripe // 16, zero_body, 0)
            pltpu.sync_copy(zbuf, shared.at[pl.ds(sid * stripe, stripe)])

            pltpu.sync_copy(vals_hbm.at[pl.ds(sid * chunk, chunk)], vals_v)
            pltpu.sync_copy(fidx_hbm.at[pl.ds(sid * chunk, chunk)], fidx_v)
            plsc.subcore_barrier()

            pltpu.sync_copy(vals_v, shared.at[fidx_v], add=True)
            plsc.subcore_barrier()

            pltpu.sync_copy(shared.at[pl.ds(sid * stripe, stripe)], zbuf)
            pltpu.sync_copy(zbuf, w_hbm.at[pl.ds(sid * stripe, stripe)])

    return build


def _conv_kernel(w3_ref, x1_ref, x2_ref, out_ref, sel_ref):
    b = pl.program_id(0)
    t = pl.program_id(1)

    @pl.when(jnp.logical_and(b == 0, t == 0))
    def _build_sel():
        iw = jax.lax.broadcasted_iota(jnp.int32, (W, W), 0)
        iq = jax.lax.broadcasted_iota(jnp.int32, (W, W), 1)
        sel_ref[...] = jnp.where(
            iq < W // 2, (iw == 2 * iq).astype(jnp.float32),
            (iw == 2 * (iq - W // 2) + 1).astype(jnp.float32))

    def row(l):
        if l < 2 * TH:
            return x1_ref[0, :, l, :]
        return x2_ref[0, :, l - 2 * TH, :]

    sel = sel_ref[...]
    dei = [jnp.dot(row(l), sel, preferred_element_type=jnp.float32)
           for l in range(2 * TH + 1)]

    for r in range(TH):
        acc = jnp.zeros((F, OW), jnp.float32)
        for j in range(KH):
            d = dei[2 * r + j]
            ev = jax.lax.slice(d, (0, 0), (C, W // 2))
            od = jax.lax.slice(d, (0, W // 2), (C, W))
            acc = acc + jnp.dot(w3_ref[j], jax.lax.slice(ev, (0, 0), (C, OW)),
                                preferred_element_type=jnp.float32)
            acc = acc + jnp.dot(w3_ref[3 + j], jax.lax.slice(od, (0, 0), (C, OW)),
                                preferred_element_type=jnp.float32)
            acc = acc + jnp.dot(w3_ref[6 + j], jax.lax.slice(ev, (0, 1), (C, OW + 1)),
                                preferred_element_type=jnp.float32)
        out_ref[0, :, r, :] = acc


def kernel(inputs, values, row_ids, col_idx):
    nnz = values.shape[0]
    nnz_pad = -(-nnz // 128) * 128
    pad = nnz_pad - nnz
    tap = col_idx // C
    cc = col_idx - tap * C
    fidx = tap * (F * C) + row_ids * C + cc
    fidx = jnp.pad(fidx, (0, pad))
    vals_p = jnp.pad(values, (0, pad))

    w_flat = _sc_build_weights(nnz_pad)(vals_p, fidx)
    w3 = w_flat.reshape(TAPS, F, C)

    grid = (B, NT)
    outT = pl.pallas_call(
        _conv_kernel,
        grid=grid,
        in_specs=[
            pl.BlockSpec((TAPS, F, C), lambda b, t: (0, 0, 0)),
            pl.BlockSpec((1, C, 2 * TH, W), lambda b, t: (b, 0, t, 0)),
            pl.BlockSpec((1, C, 8, W),
                         lambda b, t: (b, 0, jnp.minimum(4 * t + 4, 27), 0)),
        ],
        out_specs=pl.BlockSpec((1, F, TH, OW), lambda b, t: (b, 0, t, 0)),
        out_shape=jax.ShapeDtypeStruct((B, F, OH, OW), jnp.float32),
        scratch_shapes=[pltpu.VMEM((W, W), jnp.float32)],
        compiler_params=pltpu.CompilerParams(
            dimension_semantics=("arbitrary", "arbitrary")),
    )(w3, inputs, inputs)
    return jnp.swapaxes(outT, 2, 3)
